# Initial kernel scaffold; baseline (speedup 1.0000x reference)
import functools

import jax
import jax.numpy as jnp
from jax import lax
from jax.experimental import pallas as pl
from jax.experimental.pallas import tpu as pltpu
from jax.experimental.pallas import tpu_sc as plsc

D = 128
CHUNK = 128
NC = 2
NS = 16
NW = NC * NS


def _make_sc_agg(n_pad, n_chunks, with_deg):
  rows_per_tile = n_pad // NS
  zero_blocks = rows_per_tile // CHUNK
  mesh = plsc.VectorSubcoreMesh(core_axis_name="c", subcore_axis_name="s")

  out_type = [jax.ShapeDtypeStruct((NC, n_pad, D), jnp.float32)]
  if with_deg:
    out_type.append(jax.ShapeDtypeStruct((NC, n_pad), jnp.float32))

  scratch = [
      pltpu.VMEM((n_chunks, CHUNK), jnp.int32),
      pltpu.VMEM((n_chunks, CHUNK), jnp.int32),
      pltpu.VMEM((CHUNK, D), jnp.float32),
      pltpu.VMEM((CHUNK, D), jnp.float32),
      pltpu.VMEM((CHUNK,), jnp.float32),
      pltpu.VMEM_SHARED((n_pad, D), jnp.float32),
      pltpu.VMEM_SHARED((n_pad,), jnp.float32),
      pltpu.SemaphoreType.DMA,
  ]

  @functools.partial(
      pl.kernel,
      mesh=mesh,
      out_type=tuple(out_type),
      scratch_types=scratch,
  )
  def sc_agg(x_hbm, src_hbm, dst_hbm, *refs):
    if with_deg:
      sum_out, deg_out = refs[0], refs[1]
      rest = refs[2:]
    else:
      sum_out = refs[0]
      deg_out = None
      rest = refs[1:]
    src_v, dst_v, rows_v, zblk_v, ones_v, acc_sh, deg_sh, sem = rest

    c = lax.axis_index("c")
    s = lax.axis_index("s")
    wid = s * NC + c
    base = s * rows_per_tile

    def zfill(i, _):
      zblk_v[i // (D // 16), pl.ds((i % (D // 16)) * 16, 16)] = (
          jnp.zeros((16,), jnp.float32))
      return 0
    lax.fori_loop(0, CHUNK * (D // 16), zfill, 0)
    for i in range(CHUNK // 16):
      ones_v[pl.ds(i * 16, 16)] = jnp.ones((16,), jnp.float32)

    for k in range(zero_blocks):
      pltpu.sync_copy(zblk_v, acc_sh.at[pl.ds(base + k * CHUNK, CHUNK)])
    if with_deg:
      for k in range(zero_blocks):
        pltpu.sync_copy(zblk_v.at[0], deg_sh.at[pl.ds(base + k * CHUNK, CHUNK)])
    plsc.subcore_barrier()

    pltpu.sync_copy(src_hbm.at[pl.ds(wid * n_chunks, n_chunks)], src_v)
    pltpu.sync_copy(dst_hbm.at[pl.ds(wid * n_chunks, n_chunks)], dst_v)

    def body(i, _):
      pltpu.async_copy(x_hbm.at[src_v.at[i]], rows_v, sem).wait()
      pltpu.sync_copy(rows_v, acc_sh.at[dst_v.at[i]], add=True)
      if with_deg:
        pltpu.sync_copy(ones_v, deg_sh.at[dst_v.at[i]], add=True)
      return 0
    lax.fori_loop(0, n_chunks, body, 0)

    plsc.subcore_barrier()
    pltpu.sync_copy(acc_sh.at[pl.ds(base, rows_per_tile)],
                    sum_out.at[c, pl.ds(base, rows_per_tile)])
    if with_deg:
      pltpu.sync_copy(deg_sh.at[pl.ds(base, rows_per_tile)],
                      deg_out.at[c, pl.ds(base, rows_per_tile)])

  return sc_agg


def _dot(a, b):
  return jnp.dot(a, b, precision=lax.Precision.HIGHEST,
                 preferred_element_type=jnp.float32)


def _leaky(h):
  return jnp.where(h >= 0, h, 0.01 * h)


def _tc_layer1(sums, deg, xp, Wl, bl, Wr, n_pad, br=512):
  def body(sum_ref, deg_ref, x_ref, wl_ref, bl_ref, wr_ref, h_ref):
    r = pl.program_id(0)
    s = sum_ref[0] + sum_ref[1]
    dg = deg_ref[:, pl.ds(r * br, br)]
    dg = jnp.clip(dg[0] + dg[1], 1.0, None)
    mean = s / dg[:, None]
    h = _dot(mean, wl_ref[...]) + bl_ref[...] + _dot(x_ref[...], wr_ref[...])
    norm = jnp.sqrt(jnp.sum(h * h, axis=1, keepdims=True))
    h = h / jnp.clip(norm, 1e-12, None)
    h_ref[...] = _leaky(h)

  return pl.pallas_call(
      body,
      grid=(n_pad // br,),
      in_specs=[
          pl.BlockSpec((NC, br, D), lambda r: (0, r, 0)),
          pl.BlockSpec((NC, n_pad), lambda r: (0, 0)),
          pl.BlockSpec((br, D), lambda r: (r, 0)),
          pl.BlockSpec((D, D), lambda r: (0, 0)),
          pl.BlockSpec((1, D), lambda r: (0, 0)),
          pl.BlockSpec((D, D), lambda r: (0, 0)),
      ],
      out_specs=pl.BlockSpec((br, D), lambda r: (r, 0)),
      out_shape=jax.ShapeDtypeStruct((n_pad, D), jnp.float32),
  )(sums, deg, xp, Wl, bl, Wr)


def _tc_layer2(sums, deg, hp, Wl, bl, Wr, Wlin, blin, n_pad, br=512):
  def body(sum_ref, deg_ref, h_ref, wl_ref, bl_ref, wr_ref, wlin_ref,
           blin_ref, out_ref):
    r = pl.program_id(0)
    s = sum_ref[0] + sum_ref[1]
    dg = deg_ref[:, pl.ds(r * br, br)]
    dg = jnp.clip(dg[0] + dg[1], 1.0, None)
    mean = s / dg[:, None]
    h = _dot(mean, wl_ref[...]) + bl_ref[...] + _dot(h_ref[...], wr_ref[...])
    h = _leaky(h)
    out_ref[...] = _dot(h, wlin_ref[...]) + blin_ref[...]

  return pl.pallas_call(
      body,
      grid=(n_pad // br,),
      in_specs=[
          pl.BlockSpec((NC, br, D), lambda r: (0, r, 0)),
          pl.BlockSpec((NC, n_pad), lambda r: (0, 0)),
          pl.BlockSpec((br, D), lambda r: (r, 0)),
          pl.BlockSpec((D, D), lambda r: (0, 0)),
          pl.BlockSpec((1, D), lambda r: (0, 0)),
          pl.BlockSpec((D, D), lambda r: (0, 0)),
          pl.BlockSpec((D, 1), lambda r: (0, 0)),
          pl.BlockSpec((1, 1), lambda r: (0, 0)),
      ],
      out_specs=pl.BlockSpec((br, 1), lambda r: (r, 0)),
      out_shape=jax.ShapeDtypeStruct((n_pad, 1), jnp.float32),
  )(sums, deg, hp, Wl, bl, Wr, Wlin, blin)


def kernel(x, edge_index, edge_weight, Wl1, bl1, Wr1, Wl2, bl2, Wr2,
           Wlin, blin):
  del edge_weight
  n = x.shape[0]
  e = edge_index.shape[1]

  rows_per_tile = -(-(n + 1) // (NS * CHUNK)) * CHUNK
  n_pad = NS * rows_per_tile

  n_chunks = -(-e // (NW * CHUNK))
  e_pad = NW * n_chunks * CHUNK

  src = jnp.concatenate(
      [edge_index[0], jnp.zeros((e_pad - e,), jnp.int32)]).reshape(
          NW * n_chunks, CHUNK)
  dst = jnp.concatenate(
      [edge_index[1], jnp.full((e_pad - e,), n, jnp.int32)]).reshape(
          NW * n_chunks, CHUNK)

  xp = jnp.pad(x, ((0, n_pad - n), (0, 0)))

  sc_agg1 = _make_sc_agg(n_pad, n_chunks, with_deg=True)
  sc_agg2 = _make_sc_agg(n_pad, n_chunks, with_deg=False)

  sums1, deg = sc_agg1(xp, src, dst)
  h1 = _tc_layer1(sums1, deg, xp, Wl1, bl1.reshape(1, D), Wr1, n_pad)
  (sums2,) = sc_agg2(h1, src, dst)
  out = _tc_layer2(sums2, deg, h1, Wl2, bl2.reshape(1, D), Wr2,
                   Wlin, blin.reshape(1, 1), n_pad)
  return out[:n]


# speedup vs baseline: 4.8023x; 4.8023x over previous
import functools

import jax
import jax.numpy as jnp
from jax import lax
from jax.experimental import pallas as pl
from jax.experimental.pallas import tpu as pltpu
from jax.experimental.pallas import tpu_sc as plsc

D = 128
CHUNK = 128
NC = 2
NS = 16
NW = NC * NS


def _make_sc_agg(n_pad, n_chunks, with_deg):
  rows_per_tile = n_pad // NS
  zero_blocks = rows_per_tile // CHUNK
  mesh = plsc.VectorSubcoreMesh(core_axis_name="c", subcore_axis_name="s")

  out_type = [jax.ShapeDtypeStruct((NC, n_pad, D), jnp.float32)]
  if with_deg:
    out_type.append(jax.ShapeDtypeStruct((NC, n_pad), jnp.float32))

  scratch = [
      pltpu.VMEM((n_chunks, CHUNK), jnp.int32),
      pltpu.VMEM((n_chunks, CHUNK), jnp.int32),
      pltpu.VMEM((CHUNK, D), jnp.float32),
      pltpu.VMEM((CHUNK,), jnp.float32),
      pltpu.VMEM_SHARED((n_pad, D), jnp.float32),
      pltpu.VMEM_SHARED((n_pad,), jnp.float32),
      pltpu.SemaphoreType.DMA,
  ]

  @functools.partial(
      pl.kernel,
      mesh=mesh,
      out_type=tuple(out_type),
      scratch_types=scratch,
  )
  def sc_agg(x_hbm, src_hbm, dst_hbm, *refs):
    if with_deg:
      sum_out, deg_out = refs[0], refs[1]
      rest = refs[2:]
    else:
      sum_out = refs[0]
      deg_out = None
      rest = refs[1:]
    src_v, dst_v, rows_v, ones_v, acc_sh, deg_sh, sem = rest

    c = lax.axis_index("c")
    s = lax.axis_index("s")
    wid = s * NC + c
    base = s * rows_per_tile

    def zfill(i, _):
      rows_v[i // (D // 16), pl.ds((i % (D // 16)) * 16, 16)] = (
          jnp.zeros((16,), jnp.float32))
      return 0
    lax.fori_loop(0, CHUNK * (D // 16), zfill, 0)
    for i in range(CHUNK // 16):
      ones_v[pl.ds(i * 16, 16)] = jnp.ones((16,), jnp.float32)

    for k in range(zero_blocks):
      pltpu.sync_copy(rows_v, acc_sh.at[pl.ds(base + k * CHUNK, CHUNK)])
    if with_deg:
      for k in range(zero_blocks):
        pltpu.sync_copy(rows_v.at[0], deg_sh.at[pl.ds(base + k * CHUNK, CHUNK)])
    plsc.subcore_barrier()

    pltpu.sync_copy(src_hbm.at[wid], src_v)
    pltpu.sync_copy(dst_hbm.at[wid], dst_v)

    def body(i, _):
      pltpu.async_copy(x_hbm.at[src_v.at[i]], rows_v, sem).wait()
      pltpu.sync_copy(rows_v, acc_sh.at[dst_v.at[i]], add=True)
      if with_deg:
        pltpu.sync_copy(ones_v, deg_sh.at[dst_v.at[i]], add=True)
      return 0
    lax.fori_loop(0, n_chunks, body, 0)

    plsc.subcore_barrier()
    pltpu.sync_copy(acc_sh.at[pl.ds(base, rows_per_tile)],
                    sum_out.at[c, pl.ds(base, rows_per_tile)])
    if with_deg:
      pltpu.sync_copy(deg_sh.at[pl.ds(base, rows_per_tile)],
                      deg_out.at[c, pl.ds(base, rows_per_tile)])

  return sc_agg


def _dot(a, b):
  return jnp.dot(a, b, precision=lax.Precision.HIGHEST,
                 preferred_element_type=jnp.float32)


def _leaky(h):
  return jnp.where(h >= 0, h, 0.01 * h)


def _tc_layer1(sums, deg, xp, Wl, bl, Wr, n_pad, br=512):
  def body(sum_ref, deg_ref, x_ref, wl_ref, bl_ref, wr_ref, h_ref):
    r = pl.program_id(0)
    s = sum_ref[0] + sum_ref[1]
    dg = deg_ref[:, pl.ds(r * br, br)]
    dg = jnp.clip(dg[0] + dg[1], 1.0, None)
    mean = s / dg[:, None]
    h = _dot(mean, wl_ref[...]) + bl_ref[...] + _dot(x_ref[...], wr_ref[...])
    norm = jnp.sqrt(jnp.sum(h * h, axis=1, keepdims=True))
    h = h / jnp.clip(norm, 1e-12, None)
    h_ref[...] = _leaky(h)

  return pl.pallas_call(
      body,
      grid=(n_pad // br,),
      in_specs=[
          pl.BlockSpec((NC, br, D), lambda r: (0, r, 0)),
          pl.BlockSpec((NC, n_pad), lambda r: (0, 0)),
          pl.BlockSpec((br, D), lambda r: (r, 0)),
          pl.BlockSpec((D, D), lambda r: (0, 0)),
          pl.BlockSpec((1, D), lambda r: (0, 0)),
          pl.BlockSpec((D, D), lambda r: (0, 0)),
      ],
      out_specs=pl.BlockSpec((br, D), lambda r: (r, 0)),
      out_shape=jax.ShapeDtypeStruct((n_pad, D), jnp.float32),
  )(sums, deg, xp, Wl, bl, Wr)


def _tc_layer2(sums, deg, hp, Wl, bl, Wr, Wlin, blin, n_pad, br=512):
  def body(sum_ref, deg_ref, h_ref, wl_ref, bl_ref, wr_ref, wlin_ref,
           blin_ref, out_ref):
    r = pl.program_id(0)
    s = sum_ref[0] + sum_ref[1]
    dg = deg_ref[:, pl.ds(r * br, br)]
    dg = jnp.clip(dg[0] + dg[1], 1.0, None)
    mean = s / dg[:, None]
    h = _dot(mean, wl_ref[...]) + bl_ref[...] + _dot(h_ref[...], wr_ref[...])
    h = _leaky(h)
    out_ref[...] = _dot(h, wlin_ref[...]) + blin_ref[...]

  return pl.pallas_call(
      body,
      grid=(n_pad // br,),
      in_specs=[
          pl.BlockSpec((NC, br, D), lambda r: (0, r, 0)),
          pl.BlockSpec((NC, n_pad), lambda r: (0, 0)),
          pl.BlockSpec((br, D), lambda r: (r, 0)),
          pl.BlockSpec((D, D), lambda r: (0, 0)),
          pl.BlockSpec((1, D), lambda r: (0, 0)),
          pl.BlockSpec((D, D), lambda r: (0, 0)),
          pl.BlockSpec((D, 1), lambda r: (0, 0)),
          pl.BlockSpec((1, 1), lambda r: (0, 0)),
      ],
      out_specs=pl.BlockSpec((br, 1), lambda r: (r, 0)),
      out_shape=jax.ShapeDtypeStruct((n_pad, 1), jnp.float32),
  )(sums, deg, hp, Wl, bl, Wr, Wlin, blin)


def kernel(x, edge_index, edge_weight, Wl1, bl1, Wr1, Wl2, bl2, Wr2,
           Wlin, blin):
  del edge_weight
  n = x.shape[0]
  e = edge_index.shape[1]

  rows_per_tile = -(-(n + 1) // (NS * CHUNK)) * CHUNK
  n_pad = NS * rows_per_tile

  n_chunks = -(-e // (NW * CHUNK))
  e_pad = NW * n_chunks * CHUNK

  src = jnp.concatenate(
      [edge_index[0], jnp.zeros((e_pad - e,), jnp.int32)]).reshape(
          NW, n_chunks, CHUNK)
  dst = jnp.concatenate(
      [edge_index[1], jnp.full((e_pad - e,), n, jnp.int32)]).reshape(
          NW, n_chunks, CHUNK)

  xp = jnp.pad(x, ((0, n_pad - n), (0, 0)))

  sc_agg1 = _make_sc_agg(n_pad, n_chunks, with_deg=True)
  sc_agg2 = _make_sc_agg(n_pad, n_chunks, with_deg=False)

  sums1, deg = sc_agg1(xp, src, dst)
  h1 = _tc_layer1(sums1, deg, xp, Wl1, bl1.reshape(1, D), Wr1, n_pad)
  (sums2,) = sc_agg2(h1, src, dst)
  out = _tc_layer2(sums2, deg, h1, Wl2, bl2.reshape(1, D), Wr2,
                   Wlin, blin.reshape(1, 1), n_pad)
  return out[:n]
